# baseline (speedup 1.0000x reference)
import functools

import jax
import jax.numpy as jnp
from jax import lax
from jax.experimental import pallas as pl
from jax.experimental.pallas import tpu as pltpu
from jax.experimental.pallas import tpu_sc as plsc

_L = 128
_CW = 512
_HB = 40


def _prep(V, D, H, B, NW):
    n_full = V // _CW
    tail = V - n_full * _CW
    rounds = -(-n_full // NW)
    nbq = B // 128 // NW
    mesh = plsc.VectorSubcoreMesh(core_axis_name="c", subcore_axis_name="s")

    @functools.partial(
        pl.kernel,
        mesh=mesh,
        out_type=(
            jax.ShapeDtypeStruct((V * D // 128, 128), jnp.float32),
            jax.ShapeDtypeStruct((H, B // 128, 128), jnp.int32),
        ),
        scratch_types=[
            pltpu.VMEM((D, _CW), jnp.float32),
            pltpu.VMEM((D, _CW), jnp.float32),
            pltpu.VMEM((_CW * D // 128, 128), jnp.float32),
            pltpu.VMEM((_CW * D // 128, 128), jnp.float32),
            pltpu.VMEM((tail * D // 128, 128), jnp.float32),
            pltpu.VMEM((_HB, 128), jnp.int32),
            pltpu.SemaphoreType.DMA,
            pltpu.SemaphoreType.DMA,
            pltpu.SemaphoreType.DMA,
            pltpu.SemaphoreType.DMA,
        ],
        compiler_params=pltpu.CompilerParams(needs_layout_passes=False),
    )
    def prep(table_hbm, tail_hbm, idx_hbm, tlin_hbm, ilin_hbm,
             ti0, ti1, to0, to1, tailv, ibuf, si0, si1, st0, st1):
        tin = (ti0, ti1)
        tout = (to0, to1)
        s_i = (si0, si1)
        s_t = (st0, st1)
        c = lax.axis_index("c")
        s = lax.axis_index("s")
        wid = s * (NW // 16) + c
        iota = lax.iota(jnp.int32, 16)
        rpc = _CW * D // 128

        def fire_in(k, par):
            cid = k * NW + wid

            @pl.when(cid < n_full)
            def _():
                pltpu.async_copy(
                    table_hbm.at[:, pl.ds(cid * _CW, _CW)], tin[par],
                    s_i[par])

        def chunk_step(k, par):
            cid = k * NW + wid

            @pl.when(cid < n_full)
            def _():
                pltpu.make_async_copy(
                    table_hbm.at[:, pl.ds(0, _CW)], tin[par],
                    s_i[par]).wait()
                pltpu.make_async_copy(
                    tout[par], tlin_hbm.at[pl.ds(0, rpc)], s_t[par]).wait()

                @plsc.parallel_loop(0, _CW // 16, unroll=4)
                def tbody(kk):
                    v0 = kk * 16
                    vbase = (v0 + iota) * D
                    for d in range(D):
                        x = tin[par][d, pl.ds(v0, 16)]
                        flat = vbase + d
                        plsc.store_scatter(
                            tout[par], [flat >> 7, flat & 127], x)

                fire_in(k + 2, par)
                pltpu.async_copy(
                    tout[par], tlin_hbm.at[pl.ds(cid * rpc, rpc)], s_t[par])

        pltpu.async_copy(tlin_hbm.at[pl.ds(0, rpc)], to0, st0)
        pltpu.async_copy(tlin_hbm.at[pl.ds(0, rpc)], to1, st1)
        fire_in(0, 0)
        fire_in(1, 1)

        def p1_round(kk, carry):
            chunk_step(kk * 2, 0)
            chunk_step(kk * 2 + 1, 1)
            return carry

        lax.fori_loop(0, rounds // 2, p1_round, 0)
        pltpu.make_async_copy(to0, tlin_hbm.at[pl.ds(0, rpc)], st0).wait()
        pltpu.make_async_copy(to1, tlin_hbm.at[pl.ds(0, rpc)], st1).wait()

        if tail:
            @pl.when(wid == NW - 1)
            def _():
                pltpu.sync_copy(tail_hbm, tailv)
                pltpu.sync_copy(
                    tailv,
                    tlin_hbm.at[pl.ds(n_full * rpc, tail * D // 128)])

        def ib_round(i, carry):
            hb = i // nbq
            bq = (i % nbq) * NW + wid
            pltpu.sync_copy(
                idx_hbm.at[pl.ds(hb * _HB, _HB), pl.ds(bq * 128, 128)], ibuf)
            pltpu.sync_copy(ibuf, ilin_hbm.at[pl.ds(hb * _HB, _HB), bq])
            return carry

        lax.fori_loop(0, (H // _HB) * nbq, ib_round, 0)

    return prep


def _gather(V, D, H, B, NW):
    BPW = B // NW
    NQ = BPW // _L
    nblk = H // _HB
    npairs = _HB // 2
    mesh = plsc.VectorSubcoreMesh(core_axis_name="c", subcore_axis_name="s")

    @functools.partial(
        pl.kernel,
        mesh=mesh,
        out_type=jax.ShapeDtypeStruct((H, D // 8, B // 128, 8, 128),
                                      jnp.float32),
        scratch_types=[
            pltpu.VMEM((_HB, BPW), jnp.int32),
            pltpu.VMEM((BPW, D), jnp.float32),
            pltpu.VMEM((BPW, D), jnp.float32),
            pltpu.VMEM((D // 8, NQ, 8, 128), jnp.float32),
            pltpu.VMEM((D // 8, NQ, 8, 128), jnp.float32),
            pltpu.SemaphoreType.DMA,
            pltpu.SemaphoreType.DMA,
            pltpu.SemaphoreType.DMA,
            pltpu.SemaphoreType.DMA,
        ],
        compiler_params=pltpu.CompilerParams(use_tc_tiling_on_sc=False,
                                             needs_layout_passes=False),
    )
    def gath(tlin_hbm, idx_hbm, out_hbm, ib, r0, r1, t0, t1,
             sg0, sg1, so0, so1):
        rows = (r0, r1)
        tb = (t0, t1)
        s_g = (sg0, sg1)
        s_o = (so0, so1)
        c = lax.axis_index("c")
        s = lax.axis_index("s")
        wid = s * (NW // 16) + c
        b0 = wid * BPW
        bh0 = wid * NQ
        iota = lax.iota(jnp.int32, 16)
        dfull = [jnp.full((16,), d, jnp.int32) for d in range(D)]

        def fire_gathers(h_loc, pb):
            for q in range(NQ):
                pltpu.async_copy(
                    tlin_hbm.at[ib.at[h_loc, pl.ds(q * _L, _L)]],
                    rows[pb].at[pl.ds(q * _L, _L)], s_g[pb])

        def wait_gathers(h_loc, pb):
            for q in range(NQ):
                pltpu.make_async_copy(
                    tlin_hbm.at[ib.at[h_loc, pl.ds(q * _L, _L)]],
                    rows[pb].at[pl.ds(q * _L, _L)], s_g[pb]).wait()

        def wait_out(pb):
            pltpu.make_async_copy(
                tb[pb], out_hbm.at[0, :, pl.ds(bh0, NQ)], s_o[pb]).wait()

        def stage_a(h_loc, pb):
            wait_out(pb)
            fire_gathers(h_loc, pb)

        def stage_b(h_glob, h_loc, pb):
            wait_gathers(h_loc, pb)

            for q in range(NQ):
                @plsc.parallel_loop(0, _L // 16, unroll=4)
                def tbody(k, q=q):
                    bl0 = k * 16
                    bidx = q * _L + bl0 + iota
                    for d in range(D):
                        x = plsc.load_gather(rows[pb], [bidx, dfull[d]])
                        tb[pb][d // 8, q, d % 8, pl.ds(bl0, 16)] = x
            pltpu.async_copy(
                tb[pb], out_hbm.at[h_glob, :, pl.ds(bh0, NQ)], s_o[pb])

        pltpu.async_copy(out_hbm.at[0, :, pl.ds(bh0, NQ)], t0, so0)
        pltpu.async_copy(out_hbm.at[0, :, pl.ds(bh0, NQ)], t1, so1)

        def bloop(blk, carry):
            h_base = blk * _HB
            pltpu.sync_copy(
                idx_hbm.at[pl.ds(h_base, _HB), pl.ds(b0, BPW)], ib)
            stage_a(0, 0)
            stage_a(1, 1)
            stage_b(h_base, 0, 0)

            def pair(p, carry2):
                stage_a(2 * p, 0)
                stage_b(h_base + 2 * p - 1, 2 * p - 1, 1)
                stage_a(2 * p + 1, 1)
                stage_b(h_base + 2 * p, 2 * p, 0)
                return carry2

            lax.fori_loop(1, npairs, pair, 0)
            stage_b(h_base + _HB - 1, _HB - 1, 1)
            return carry

        lax.fori_loop(0, nblk, bloop, 0)
        wait_out(0)
        wait_out(1)

    return gath


def kernel(batch, table):
    B, H = batch.shape
    V, D = table.shape
    info = plsc.get_sparse_core_info()
    NW = info.num_cores * info.num_subcores
    batch_t = batch.T.astype(jnp.int32)
    table_t = table.T
    ntail = V % _CW
    tail16 = table[V - ntail:, :].reshape(ntail * D // 128, 128)
    tlin128, ilin3 = _prep(V, D, H, B, NW)(table_t, tail16, batch_t)
    tlin = tlin128.reshape(V, D)
    ilin = ilin3.reshape(H, B)
    out5 = _gather(V, D, H, B, NW)(tlin, ilin)
    out = out5.transpose(2, 4, 0, 1, 3)
    return out.reshape(B, H, D)


# speedup vs baseline: 1.0655x; 1.0655x over previous
import functools

import jax
import jax.numpy as jnp
from jax import lax
from jax.experimental import pallas as pl
from jax.experimental.pallas import tpu as pltpu
from jax.experimental.pallas import tpu_sc as plsc

_L = 128
_CW = 512
_HB = 40


def _prep(V, D, H, B, NW):
    n_full = V // _CW
    tail = V - n_full * _CW
    rounds = -(-n_full // NW)
    nbq = B // 128 // NW
    mesh = plsc.VectorSubcoreMesh(core_axis_name="c", subcore_axis_name="s")

    @functools.partial(
        pl.kernel,
        mesh=mesh,
        out_type=(
            jax.ShapeDtypeStruct((V * D // 128, 128), jnp.float32),
            jax.ShapeDtypeStruct((H, B // 128, 128), jnp.int32),
        ),
        scratch_types=[
            pltpu.VMEM((D, _CW), jnp.float32),
            pltpu.VMEM((D, _CW), jnp.float32),
            pltpu.VMEM((_CW * D // 128, 128), jnp.float32),
            pltpu.VMEM((_CW * D // 128, 128), jnp.float32),
            pltpu.VMEM((tail * D // 128, 128), jnp.float32),
            pltpu.VMEM((_HB, 128), jnp.int32),
            pltpu.SemaphoreType.DMA,
            pltpu.SemaphoreType.DMA,
            pltpu.SemaphoreType.DMA,
            pltpu.SemaphoreType.DMA,
        ],
        compiler_params=pltpu.CompilerParams(needs_layout_passes=False),
    )
    def prep(table_hbm, tail_hbm, idx_hbm, tlin_hbm, ilin_hbm,
             ti0, ti1, to0, to1, tailv, ibuf, si0, si1, st0, st1):
        tin = (ti0, ti1)
        tout = (to0, to1)
        s_i = (si0, si1)
        s_t = (st0, st1)
        c = lax.axis_index("c")
        s = lax.axis_index("s")
        wid = s * (NW // 16) + c
        iota = lax.iota(jnp.int32, 16)
        rpc = _CW * D // 128

        def fire_in(k, par):
            cid = k * NW + wid

            @pl.when(cid < n_full)
            def _():
                pltpu.async_copy(
                    table_hbm.at[:, pl.ds(cid * _CW, _CW)], tin[par],
                    s_i[par])

        def chunk_step(k, par):
            cid = k * NW + wid

            @pl.when(cid < n_full)
            def _():
                pltpu.make_async_copy(
                    table_hbm.at[:, pl.ds(0, _CW)], tin[par],
                    s_i[par]).wait()
                pltpu.make_async_copy(
                    tout[par], tlin_hbm.at[pl.ds(0, rpc)], s_t[par]).wait()

                @plsc.parallel_loop(0, _CW // 16, unroll=4)
                def tbody(kk):
                    v0 = kk * 16
                    vbase = (v0 + iota) * D
                    for d in range(D):
                        x = tin[par][d, pl.ds(v0, 16)]
                        flat = vbase + d
                        plsc.store_scatter(
                            tout[par], [flat >> 7, flat & 127], x)

                fire_in(k + 2, par)
                pltpu.async_copy(
                    tout[par], tlin_hbm.at[pl.ds(cid * rpc, rpc)], s_t[par])

        pltpu.async_copy(tlin_hbm.at[pl.ds(0, rpc)], to0, st0)
        pltpu.async_copy(tlin_hbm.at[pl.ds(0, rpc)], to1, st1)
        fire_in(0, 0)
        fire_in(1, 1)

        def p1_round(kk, carry):
            chunk_step(kk * 2, 0)
            chunk_step(kk * 2 + 1, 1)
            return carry

        lax.fori_loop(0, rounds // 2, p1_round, 0)
        pltpu.make_async_copy(to0, tlin_hbm.at[pl.ds(0, rpc)], st0).wait()
        pltpu.make_async_copy(to1, tlin_hbm.at[pl.ds(0, rpc)], st1).wait()

        if tail:
            @pl.when(wid == NW - 1)
            def _():
                pltpu.sync_copy(tail_hbm, tailv)
                pltpu.sync_copy(
                    tailv,
                    tlin_hbm.at[pl.ds(n_full * rpc, tail * D // 128)])

        def ib_round(i, carry):
            hb = i // nbq
            bq = (i % nbq) * NW + wid
            pltpu.sync_copy(
                idx_hbm.at[pl.ds(hb * _HB, _HB), pl.ds(bq * 128, 128)], ibuf)
            pltpu.sync_copy(ibuf, ilin_hbm.at[pl.ds(hb * _HB, _HB), bq])
            return carry

        lax.fori_loop(0, (H // _HB) * nbq, ib_round, 0)

    return prep


def _gather(V, D, H, B, NW):
    BPW = B // NW
    NQ = BPW // _L
    nblk = H // _HB
    npairs = _HB // 2
    mesh = plsc.VectorSubcoreMesh(core_axis_name="c", subcore_axis_name="s")

    @functools.partial(
        pl.kernel,
        mesh=mesh,
        out_type=jax.ShapeDtypeStruct((H, D // 8, B // 128, 8, 128),
                                      jnp.float32),
        scratch_types=[
            pltpu.VMEM((_HB, BPW), jnp.int32),
            pltpu.VMEM((BPW, D), jnp.float32),
            pltpu.VMEM((BPW, D), jnp.float32),
            pltpu.VMEM((D // 8, NQ, 8, 128), jnp.float32),
            pltpu.VMEM((D // 8, NQ, 8, 128), jnp.float32),
            pltpu.SemaphoreType.DMA,
            pltpu.SemaphoreType.DMA,
            pltpu.SemaphoreType.DMA,
            pltpu.SemaphoreType.DMA,
        ],
        compiler_params=pltpu.CompilerParams(use_tc_tiling_on_sc=False,
                                             needs_layout_passes=False),
    )
    def gath(tlin_hbm, idx_hbm, out_hbm, ib, r0, r1, t0, t1,
             sg0, sg1, so0, so1):
        rows = (r0, r1)
        tb = (t0, t1)
        s_g = (sg0, sg1)
        s_o = (so0, so1)
        c = lax.axis_index("c")
        s = lax.axis_index("s")
        wid = s * (NW // 16) + c
        b0 = wid * BPW
        bh0 = wid * NQ
        iota = lax.iota(jnp.int32, 16)
        dfull = [jnp.full((16,), d, jnp.int32) for d in range(D)]

        def fire_gathers(h_loc, pb):
            for q in range(NQ):
                pltpu.async_copy(
                    tlin_hbm.at[ib.at[h_loc, pl.ds(q * _L, _L)]],
                    rows[pb].at[pl.ds(q * _L, _L)], s_g[pb])

        def wait_gathers(h_loc, pb):
            for q in range(NQ):
                pltpu.make_async_copy(
                    tlin_hbm.at[ib.at[h_loc, pl.ds(q * _L, _L)]],
                    rows[pb].at[pl.ds(q * _L, _L)], s_g[pb]).wait()

        def wait_out(pb):
            pltpu.make_async_copy(
                tb[pb], out_hbm.at[0, :, pl.ds(bh0, NQ)], s_o[pb]).wait()

        def stage_a(h_loc, pb):
            wait_out(pb)
            fire_gathers(h_loc, pb)

        def stage_b(h_glob, h_loc, pb):
            wait_gathers(h_loc, pb)

            for q in range(NQ):
                @plsc.parallel_loop(0, _L // 16, unroll=2)
                def tbody(k, q=q):
                    bl0 = k * 16
                    bidx = q * _L + bl0 + iota
                    for d in range(D):
                        x = plsc.load_gather(rows[pb], [bidx, dfull[d]])
                        tb[pb][d // 8, q, d % 8, pl.ds(bl0, 16)] = x
            pltpu.async_copy(
                tb[pb], out_hbm.at[h_glob, :, pl.ds(bh0, NQ)], s_o[pb])

        pltpu.async_copy(out_hbm.at[0, :, pl.ds(bh0, NQ)], t0, so0)
        pltpu.async_copy(out_hbm.at[0, :, pl.ds(bh0, NQ)], t1, so1)

        def bloop(blk, carry):
            h_base = blk * _HB
            pltpu.sync_copy(
                idx_hbm.at[pl.ds(h_base, _HB), pl.ds(b0, BPW)], ib)
            stage_a(0, 0)
            stage_a(1, 1)
            stage_b(h_base, 0, 0)

            def pair(p, carry2):
                stage_a(2 * p, 0)
                stage_b(h_base + 2 * p - 1, 2 * p - 1, 1)
                stage_a(2 * p + 1, 1)
                stage_b(h_base + 2 * p, 2 * p, 0)
                return carry2

            lax.fori_loop(1, npairs, pair, 0)
            stage_b(h_base + _HB - 1, _HB - 1, 1)
            return carry

        lax.fori_loop(0, nblk, bloop, 0)
        wait_out(0)
        wait_out(1)

    return gath


def kernel(batch, table):
    B, H = batch.shape
    V, D = table.shape
    info = plsc.get_sparse_core_info()
    NW = info.num_cores * info.num_subcores
    batch_t = batch.T.astype(jnp.int32)
    table_t = table.T
    ntail = V % _CW
    tail16 = table[V - ntail:, :].reshape(ntail * D // 128, 128)
    tlin128, ilin3 = _prep(V, D, H, B, NW)(table_t, tail16, batch_t)
    tlin = tlin128.reshape(V, D)
    ilin = ilin3.reshape(H, B)
    out5 = _gather(V, D, H, B, NW)(tlin, ilin)
    out = out5.transpose(2, 4, 0, 1, 3)
    return out.reshape(B, H, D)


# speedup vs baseline: 1.1247x; 1.0555x over previous
import functools

import jax
import jax.numpy as jnp
from jax import lax
from jax.experimental import pallas as pl
from jax.experimental.pallas import tpu as pltpu
from jax.experimental.pallas import tpu_sc as plsc

_L = 128
_CW = 512
_HB = 40


def _prep(V, D, H, B, NW):
    n_full = V // _CW
    tail = V - n_full * _CW
    rounds = -(-n_full // NW)
    nbq = B // 128 // NW
    mesh = plsc.VectorSubcoreMesh(core_axis_name="c", subcore_axis_name="s")

    @functools.partial(
        pl.kernel,
        mesh=mesh,
        out_type=(
            jax.ShapeDtypeStruct((V * D // 128, 128), jnp.float32),
            jax.ShapeDtypeStruct((H, B // 128, 128), jnp.int32),
        ),
        scratch_types=[
            pltpu.VMEM((D, _CW), jnp.float32),
            pltpu.VMEM((D, _CW), jnp.float32),
            pltpu.VMEM((_CW * D // 128, 128), jnp.float32),
            pltpu.VMEM((_CW * D // 128, 128), jnp.float32),
            pltpu.VMEM((tail * D // 128, 128), jnp.float32),
            pltpu.VMEM((_HB, 128), jnp.int32),
            pltpu.SemaphoreType.DMA,
            pltpu.SemaphoreType.DMA,
            pltpu.SemaphoreType.DMA,
            pltpu.SemaphoreType.DMA,
        ],
        compiler_params=pltpu.CompilerParams(needs_layout_passes=False),
    )
    def prep(table_hbm, tail_hbm, idx_hbm, tlin_hbm, ilin_hbm,
             ti0, ti1, to0, to1, tailv, ibuf, si0, si1, st0, st1):
        tin = (ti0, ti1)
        tout = (to0, to1)
        s_i = (si0, si1)
        s_t = (st0, st1)
        c = lax.axis_index("c")
        s = lax.axis_index("s")
        wid = s * (NW // 16) + c
        iota = lax.iota(jnp.int32, 16)
        rpc = _CW * D // 128

        def fire_in(k, par):
            cid = k * NW + wid

            @pl.when(cid < n_full)
            def _():
                pltpu.async_copy(
                    table_hbm.at[:, pl.ds(cid * _CW, _CW)], tin[par],
                    s_i[par])

        def chunk_step(k, par):
            cid = k * NW + wid

            @pl.when(cid < n_full)
            def _():
                pltpu.make_async_copy(
                    table_hbm.at[:, pl.ds(0, _CW)], tin[par],
                    s_i[par]).wait()
                pltpu.make_async_copy(
                    tout[par], tlin_hbm.at[pl.ds(0, rpc)], s_t[par]).wait()

                @plsc.parallel_loop(0, _CW // 16, unroll=4)
                def tbody(kk):
                    v0 = kk * 16
                    vbase = (v0 + iota) * D
                    for d in range(D):
                        x = tin[par][d, pl.ds(v0, 16)]
                        flat = vbase + d
                        plsc.store_scatter(
                            tout[par], [flat >> 7, flat & 127], x)

                fire_in(k + 2, par)
                pltpu.async_copy(
                    tout[par], tlin_hbm.at[pl.ds(cid * rpc, rpc)], s_t[par])

        pltpu.async_copy(tlin_hbm.at[pl.ds(0, rpc)], to0, st0)
        pltpu.async_copy(tlin_hbm.at[pl.ds(0, rpc)], to1, st1)
        fire_in(0, 0)
        fire_in(1, 1)

        def p1_round(kk, carry):
            chunk_step(kk * 2, 0)
            chunk_step(kk * 2 + 1, 1)
            return carry

        lax.fori_loop(0, rounds // 2, p1_round, 0)
        pltpu.make_async_copy(to0, tlin_hbm.at[pl.ds(0, rpc)], st0).wait()
        pltpu.make_async_copy(to1, tlin_hbm.at[pl.ds(0, rpc)], st1).wait()

        if tail:
            @pl.when(wid == NW - 1)
            def _():
                pltpu.sync_copy(tail_hbm, tailv)
                pltpu.sync_copy(
                    tailv,
                    tlin_hbm.at[pl.ds(n_full * rpc, tail * D // 128)])

        def ib_round(i, carry):
            hb = i // nbq
            bq = (i % nbq) * NW + wid
            pltpu.sync_copy(
                idx_hbm.at[pl.ds(hb * _HB, _HB), pl.ds(bq * 128, 128)], ibuf)
            pltpu.sync_copy(ibuf, ilin_hbm.at[pl.ds(hb * _HB, _HB), bq])
            return carry

        lax.fori_loop(0, (H // _HB) * nbq, ib_round, 0)

    return prep


def _gather(V, D, H, B, NW):
    BPW = B // NW
    NQ = BPW // _L
    nblk = H // _HB
    npairs = _HB // 2
    mesh = plsc.VectorSubcoreMesh(core_axis_name="c", subcore_axis_name="s")

    @functools.partial(
        pl.kernel,
        mesh=mesh,
        out_type=jax.ShapeDtypeStruct((H, D // 8, B // 128, 8, 128),
                                      jnp.float32),
        scratch_types=[
            pltpu.VMEM((_HB, BPW), jnp.int32),
            pltpu.VMEM((BPW, D), jnp.float32),
            pltpu.VMEM((BPW, D), jnp.float32),
            pltpu.VMEM((D // 8, NQ, 8, 128), jnp.float32),
            pltpu.VMEM((D // 8, NQ, 8, 128), jnp.float32),
            pltpu.SemaphoreType.DMA,
            pltpu.SemaphoreType.DMA,
            pltpu.SemaphoreType.DMA,
            pltpu.SemaphoreType.DMA,
        ],
        compiler_params=pltpu.CompilerParams(use_tc_tiling_on_sc=False,
                                             needs_layout_passes=False),
    )
    def gath(tlin_hbm, idx_hbm, out_hbm, ib, r0, r1, t0, t1,
             sg0, sg1, so0, so1):
        rows = (r0, r1)
        tb = (t0, t1)
        s_g = (sg0, sg1)
        s_o = (so0, so1)
        c = lax.axis_index("c")
        s = lax.axis_index("s")
        wid = s * (NW // 16) + c
        b0 = wid * BPW
        bh0 = wid * NQ
        iota = lax.iota(jnp.int32, 16)
        dfull = [jnp.full((16,), d, jnp.int32) for d in range(D)]

        def fire_gathers(h_loc, pb):
            for q in range(NQ):
                pltpu.async_copy(
                    tlin_hbm.at[ib.at[h_loc, pl.ds(q * _L, _L)]],
                    rows[pb].at[pl.ds(q * _L, _L)], s_g[pb])

        def wait_gathers(h_loc, pb):
            for q in range(NQ):
                pltpu.make_async_copy(
                    tlin_hbm.at[ib.at[h_loc, pl.ds(q * _L, _L)]],
                    rows[pb].at[pl.ds(q * _L, _L)], s_g[pb]).wait()

        def wait_out(pb):
            pltpu.make_async_copy(
                tb[pb], out_hbm.at[0, :, pl.ds(bh0, NQ)], s_o[pb]).wait()

        def stage_a(h_loc, pb):
            fire_gathers(h_loc, pb)

        def stage_b(h_glob, h_loc, pb):
            wait_gathers(h_loc, pb)
            wait_out(pb)

            for q in range(NQ):
                @plsc.parallel_loop(0, _L // 16, unroll=2)
                def tbody(k, q=q):
                    bl0 = k * 16
                    bidx = q * _L + bl0 + iota
                    for d in range(D):
                        x = plsc.load_gather(rows[pb], [bidx, dfull[d]])
                        tb[pb][d // 8, q, d % 8, pl.ds(bl0, 16)] = x
            pltpu.async_copy(
                tb[pb], out_hbm.at[h_glob, :, pl.ds(bh0, NQ)], s_o[pb])

        pltpu.async_copy(out_hbm.at[0, :, pl.ds(bh0, NQ)], t0, so0)
        pltpu.async_copy(out_hbm.at[0, :, pl.ds(bh0, NQ)], t1, so1)

        def bloop(blk, carry):
            h_base = blk * _HB
            pltpu.sync_copy(
                idx_hbm.at[pl.ds(h_base, _HB), pl.ds(b0, BPW)], ib)
            stage_a(0, 0)
            stage_a(1, 1)
            stage_b(h_base, 0, 0)

            def pair(p, carry2):
                stage_a(2 * p, 0)
                stage_b(h_base + 2 * p - 1, 2 * p - 1, 1)
                stage_a(2 * p + 1, 1)
                stage_b(h_base + 2 * p, 2 * p, 0)
                return carry2

            lax.fori_loop(1, npairs, pair, 0)
            stage_b(h_base + _HB - 1, _HB - 1, 1)
            return carry

        lax.fori_loop(0, nblk, bloop, 0)
        wait_out(0)
        wait_out(1)

    return gath


def kernel(batch, table):
    B, H = batch.shape
    V, D = table.shape
    info = plsc.get_sparse_core_info()
    NW = info.num_cores * info.num_subcores
    batch_t = batch.T.astype(jnp.int32)
    table_t = table.T
    ntail = V % _CW
    tail16 = table[V - ntail:, :].reshape(ntail * D // 128, 128)
    tlin128, ilin3 = _prep(V, D, H, B, NW)(table_t, tail16, batch_t)
    tlin = tlin128.reshape(V, D)
    ilin = ilin3.reshape(H, B)
    out5 = _gather(V, D, H, B, NW)(tlin, ilin)
    out = out5.transpose(2, 4, 0, 1, 3)
    return out.reshape(B, H, D)
